# Initial kernel scaffold; baseline (speedup 1.0000x reference)
import jax
import jax.numpy as jnp
from jax.experimental import pallas as pl


def kernel(node_feats_seq, edge_index_seq, prev_pos_seq, W_gcn1, b_gcn1, W_gcn2, b_gcn2, W_ih, W_hh, b_ih, b_hh, W_m1, b_m1, W_m2, b_m2, W_m3, b_m3):
    raise NotImplementedError("write your pallas kernel here")


# speedup vs baseline: 35.3168x; 35.3168x over previous
import functools

import jax
import jax.numpy as jnp
from jax import lax
from jax.experimental import pallas as pl
from jax.experimental.pallas import tpu as pltpu
from jax.experimental.pallas import tpu_sc as plsc

B, T, N, F = 2, 4, 10000, 16
GH = LH = MH = 128
G = B * T
E = 160000
NPAD = 112
NE = N + NPAD
NT = 16
CH = 128
NCH = 79
EP = NT * NCH * CH
GPC = G // 2
ROWS = GPC * NE // NT


def _sc_mesh():
    return plsc.VectorSubcoreMesh(core_axis_name="c", subcore_axis_name="s")


def _deg_body(dstl_hbm, zvec_hbm, deg_out, idx_v, ones_v, stage_v, deg_sh,
              sem):
    c = lax.axis_index("c")
    s = lax.axis_index("s")
    pltpu.sync_copy(zvec_hbm.at[pl.ds(s * ROWS, ROWS)], stage_v)
    pltpu.sync_copy(stage_v, deg_sh.at[pl.ds(s * ROWS, ROWS)])
    for i in range(CH // 16):
        ones_v[pl.ds(i * 16, 16)] = jnp.ones((16,), jnp.float32)
    plsc.subcore_barrier()
    for g in range(GPC):
        ga = c * GPC + g
        pltpu.sync_copy(dstl_hbm.at[ga].at[s], idx_v)

        def chunk(j, carry):
            pltpu.sync_copy(ones_v, deg_sh.at[idx_v.at[j]], add=True)
            return carry

        lax.fori_loop(0, NCH, chunk, 0)
    plsc.subcore_barrier()
    pltpu.sync_copy(deg_sh.at[pl.ds(s * ROWS, ROWS)], stage_v)
    pltpu.sync_copy(stage_v,
                    deg_out.at[pl.ds(c * GPC * NE + s * ROWS, ROWS)])


def _degrees(dst_l, z_vec):
    return pl.kernel(
        _deg_body,
        out_type=jax.ShapeDtypeStruct((G * NE,), jnp.float32),
        mesh=_sc_mesh(),
        scratch_types=[
            pltpu.VMEM((NCH, CH), jnp.int32),
            pltpu.VMEM((CH,), jnp.float32),
            pltpu.VMEM((ROWS,), jnp.float32),
            pltpu.VMEM_SHARED((GPC * NE,), jnp.float32),
            pltpu.SemaphoreType.DMA,
        ],
    )(dst_l, z_vec)


def _gcn_body(xs_hbm, dinv_hbm, srcg_hbm, dstg_hbm, srcl_hbm, dstl_hbm,
              zacc_hbm, zvec_hbm, acc_out, osum_out,
              sg_v, dg_v, sl_v, dl_v, rows_v, dval_v, stage_v, stagev_v,
              acc_sh, osum_sh, sem):
    c = lax.axis_index("c")
    s = lax.axis_index("s")
    pltpu.sync_copy(zacc_hbm.at[pl.ds(s * ROWS, ROWS)], stage_v)
    pltpu.sync_copy(stage_v, acc_sh.at[pl.ds(s * ROWS, ROWS)])
    pltpu.sync_copy(zvec_hbm.at[pl.ds(s * ROWS, ROWS)], stagev_v)
    pltpu.sync_copy(stagev_v, osum_sh.at[pl.ds(s * ROWS, ROWS)])
    plsc.subcore_barrier()
    for g in range(GPC):
        ga = c * GPC + g
        pltpu.sync_copy(srcg_hbm.at[ga].at[s], sg_v)
        pltpu.sync_copy(dstg_hbm.at[ga].at[s], dg_v)
        pltpu.sync_copy(srcl_hbm.at[ga].at[s], sl_v)
        pltpu.sync_copy(dstl_hbm.at[ga].at[s], dl_v)

        def chunk(j, carry):
            pltpu.async_copy(xs_hbm.at[sg_v.at[j]], rows_v, sem).wait()
            pltpu.sync_copy(rows_v, acc_sh.at[dl_v.at[j]], add=True)
            pltpu.async_copy(dinv_hbm.at[dg_v.at[j]], dval_v, sem).wait()
            pltpu.sync_copy(dval_v, osum_sh.at[sl_v.at[j]], add=True)
            return carry

        lax.fori_loop(0, NCH, chunk, 0)
    plsc.subcore_barrier()
    base = c * GPC * NE + s * ROWS
    pltpu.sync_copy(acc_sh.at[pl.ds(s * ROWS, ROWS)], stage_v)
    pltpu.sync_copy(stage_v, acc_out.at[pl.ds(base, ROWS)])
    pltpu.sync_copy(osum_sh.at[pl.ds(s * ROWS, ROWS)], stagev_v)
    pltpu.sync_copy(stagev_v, osum_out.at[pl.ds(base, ROWS)])


def _message_passing(xs_rows, dinv_flat, src_g, dst_g, src_l, dst_l,
                     z_acc, z_vec):
    return pl.kernel(
        _gcn_body,
        out_type=(jax.ShapeDtypeStruct((G * NE, F), jnp.float32),
                  jax.ShapeDtypeStruct((G * NE,), jnp.float32)),
        mesh=_sc_mesh(),
        scratch_types=[
            pltpu.VMEM((NCH, CH), jnp.int32),
            pltpu.VMEM((NCH, CH), jnp.int32),
            pltpu.VMEM((NCH, CH), jnp.int32),
            pltpu.VMEM((NCH, CH), jnp.int32),
            pltpu.VMEM((CH, F), jnp.float32),
            pltpu.VMEM((CH,), jnp.float32),
            pltpu.VMEM((ROWS, F), jnp.float32),
            pltpu.VMEM((ROWS,), jnp.float32),
            pltpu.VMEM_SHARED((GPC * NE, F), jnp.float32),
            pltpu.VMEM_SHARED((GPC * NE,), jnp.float32),
            pltpu.SemaphoreType.DMA,
        ],
        compiler_params=pltpu.CompilerParams(use_tc_tiling_on_sc=False),
    )(xs_rows, dinv_flat, src_g, dst_g, src_l, dst_l, z_acc, z_vec)


def _prescale_body(deg_ref, x_ref, dinv_ref, xs_ref):
    di = lax.rsqrt(deg_ref[...] + 1.0)
    dinv_ref[...] = di
    dit = jnp.transpose(di[0])
    xs_ref[...] = x_ref[...] * dit[None]


def _prescale(deg3, x3):
    return pl.pallas_call(
        _prescale_body,
        grid=(G,),
        in_specs=[
            pl.BlockSpec((1, 1, NE), lambda g: (g, 0, 0)),
            pl.BlockSpec((1, NE, F), lambda g: (g, 0, 0)),
        ],
        out_specs=[
            pl.BlockSpec((1, 1, NE), lambda g: (g, 0, 0)),
            pl.BlockSpec((1, NE, F), lambda g: (g, 0, 0)),
        ],
        out_shape=(jax.ShapeDtypeStruct((G, 1, NE), jnp.float32),
                   jax.ShapeDtypeStruct((G, NE, F), jnp.float32)),
        compiler_params=pltpu.CompilerParams(
            dimension_semantics=("arbitrary",)),
    )(deg3, x3)


def _final_body(acc_ref, xs_ref, dinv_ref, osum_ref, w1_ref, b1_ref,
                w2_ref, b2_ref, wig_ref, wip_ref, whh_ref, bih_ref, bhh_ref,
                prev_ref, wm1_ref, bm1_ref, wm2_ref, bm2_ref, wm3_ref,
                bm3_ref, out_ref, ge_s):
    g = pl.program_id(0)
    di = dinv_ref[0]
    z = jnp.dot(acc_ref[0] + xs_ref[0], w1_ref[...],
                preferred_element_type=jnp.float32)
    h1 = jnp.maximum(jnp.transpose(di) * z + b1_ref[...], 0.0)
    mask = lax.broadcasted_iota(jnp.int32, (1, NE), 1) < N
    coef = jnp.where(mask, di * osum_ref[0] + di * di, 0.0)
    sv = jnp.dot(coef, h1, preferred_element_type=jnp.float32)
    pooled = b2_ref[...] + jnp.dot(sv, w2_ref[...],
                                   preferred_element_type=jnp.float32) / N
    ge_s[pl.ds(g, 1), :] = pooled

    @pl.when(g == G - 1)
    def _():
        ps = ge_s[...]
        h = jnp.zeros((B, LH), jnp.float32)
        cc = jnp.zeros((B, LH), jnp.float32)
        for t in range(T):
            ge_t = jnp.concatenate([ps[t:t + 1], ps[T + t:T + t + 1]], axis=0)
            pp = prev_ref[:, t, :]
            gates = (jnp.dot(ge_t, wig_ref[...],
                             preferred_element_type=jnp.float32)
                     + jnp.dot(pp, wip_ref[...],
                               preferred_element_type=jnp.float32)
                     + jnp.dot(h, whh_ref[...],
                               preferred_element_type=jnp.float32)
                     + bih_ref[...] + bhh_ref[...])
            ig = jax.nn.sigmoid(gates[:, :LH])
            fg = jax.nn.sigmoid(gates[:, LH:2 * LH])
            gg = jnp.tanh(gates[:, 2 * LH:3 * LH])
            og = jax.nn.sigmoid(gates[:, 3 * LH:])
            cc = fg * cc + ig * gg
            h = og * jnp.tanh(cc)
        z1 = jnp.maximum(jnp.dot(h, wm1_ref[...],
                                 preferred_element_type=jnp.float32)
                         + bm1_ref[...], 0.0)
        z2 = jnp.maximum(jnp.dot(z1, wm2_ref[...],
                                 preferred_element_type=jnp.float32)
                         + bm2_ref[...], 0.0)
        out_ref[...] = (jnp.dot(z2, wm3_ref[...],
                                preferred_element_type=jnp.float32)
                        + bm3_ref[...])


def _finalize(acc3, xs3, dinv3, osum3, w1, b1, w2, b2, wig, wip, whh,
              bih, bhh, prev, wm1, bm1, wm2, bm2, wm3, bm3):
    full = lambda shape: pl.BlockSpec(shape, lambda g: tuple(0 for _ in shape))
    return pl.pallas_call(
        _final_body,
        grid=(G,),
        in_specs=[
            pl.BlockSpec((1, NE, F), lambda g: (g, 0, 0)),
            pl.BlockSpec((1, NE, F), lambda g: (g, 0, 0)),
            pl.BlockSpec((1, 1, NE), lambda g: (g, 0, 0)),
            pl.BlockSpec((1, 1, NE), lambda g: (g, 0, 0)),
            full((F, GH)), full((1, GH)),
            full((GH, GH)), full((1, GH)),
            full((GH, 4 * LH)), full((3, 4 * LH)), full((LH, 4 * LH)),
            full((1, 4 * LH)), full((1, 4 * LH)),
            full((B, T, 3)),
            full((LH, MH)), full((1, MH)),
            full((MH, MH)), full((1, MH)),
            full((MH, 3)), full((1, 3)),
        ],
        out_specs=pl.BlockSpec((B, 3), lambda g: (0, 0)),
        out_shape=jax.ShapeDtypeStruct((B, 3), jnp.float32),
        scratch_shapes=[pltpu.VMEM((G, GH), jnp.float32)],
        compiler_params=pltpu.CompilerParams(
            dimension_semantics=("arbitrary",)),
    )(acc3, xs3, dinv3, osum3, w1, b1, w2, b2, wig, wip, whh, bih, bhh,
      prev, wm1, bm1, wm2, bm2, wm3, bm3)


def kernel(node_feats_seq, edge_index_seq, prev_pos_seq,
           W_gcn1, b_gcn1, W_gcn2, b_gcn2,
           W_ih, W_hh, b_ih, b_hh,
           W_m1, b_m1, W_m2, b_m2, W_m3, b_m3):
    f32 = jnp.float32
    ei = edge_index_seq.astype(jnp.int32).reshape(G, 2, E)
    pad = N + (jnp.arange(EP - E, dtype=jnp.int32) % NPAD)
    pad = jnp.broadcast_to(pad, (G, EP - E))
    src = jnp.concatenate([ei[:, 0], pad], axis=1)
    dst = jnp.concatenate([ei[:, 1], pad], axis=1)
    goff = (jnp.arange(G, dtype=jnp.int32) * NE)[:, None]
    loff = ((jnp.arange(G, dtype=jnp.int32) % GPC) * NE)[:, None]
    shp = (G, NT, NCH, CH)
    src_g = (src + goff).reshape(shp)
    dst_g = (dst + goff).reshape(shp)
    src_l = (src + loff).reshape(shp)
    dst_l = (dst + loff).reshape(shp)

    z_vec = jnp.zeros((GPC * NE,), f32)
    z_acc = jnp.zeros((GPC * NE, F), f32)

    deg = _degrees(dst_l, z_vec)

    x3 = jnp.pad(node_feats_seq.reshape(G, N, F), ((0, 0), (0, NPAD), (0, 0)))
    dinv3, xs3 = _prescale(deg.reshape(G, 1, NE), x3)

    acc, osum = _message_passing(
        xs3.reshape(G * NE, F), dinv3.reshape(G * NE),
        src_g, dst_g, src_l, dst_l, z_acc, z_vec)

    wih_t = jnp.transpose(W_ih)
    pred = _finalize(
        acc.reshape(G, NE, F), xs3, dinv3, osum.reshape(G, 1, NE),
        W_gcn1, b_gcn1.reshape(1, GH), W_gcn2, b_gcn2.reshape(1, GH),
        wih_t[:GH], wih_t[GH:], jnp.transpose(W_hh),
        b_ih.reshape(1, 4 * LH), b_hh.reshape(1, 4 * LH),
        prev_pos_seq,
        jnp.transpose(W_m1), b_m1.reshape(1, MH),
        jnp.transpose(W_m2), b_m2.reshape(1, MH),
        jnp.transpose(W_m3), b_m3.reshape(1, 3))
    return pred


# speedup vs baseline: 79.7745x; 2.2588x over previous
import functools

import jax
import jax.numpy as jnp
from jax import lax
from jax.experimental import pallas as pl
from jax.experimental.pallas import tpu as pltpu
from jax.experimental.pallas import tpu_sc as plsc

B, T, N, F = 2, 4, 10000, 16
GH = LH = MH = 128
G = B * T
E = 160000
NPAD = 112
NE = N + NPAD
NT = 16
CH = 128
NCH = 79
EP = NT * NCH * CH
GPC = G // 2
PG = 2
NPASS = GPC // PG
PR = PG * NE // NT


def _sc_mesh():
    return plsc.VectorSubcoreMesh(core_axis_name="c", subcore_axis_name="s")


def _deg_body(dstl_hbm, deg_out, idx_v, idx2_v, ones_v, stage_v, deg_sh,
              isem, ssem):
    c = lax.axis_index("c")
    s = lax.axis_index("s")
    for i in range(CH // 16):
        ones_v[pl.ds(i * 16, 16)] = jnp.ones((16,), jnp.float32)
    bufs = [idx_v, idx2_v]
    for p in range(NPASS):
        for i in range(PR // 16):
            stage_v[pl.ds(i * 16, 16)] = jnp.zeros((16,), jnp.float32)
        pltpu.sync_copy(stage_v, deg_sh.at[pl.ds(s * PR, PR)])
        plsc.subcore_barrier()
        pltpu.async_copy(dstl_hbm.at[c * GPC + p * PG].at[s], bufs[0], isem)
        for g in range(PG):
            ib = bufs[g % 2]
            pltpu.make_async_copy(dstl_hbm.at[0].at[s], ib, isem).wait()
            if g + 1 < PG:
                pltpu.async_copy(dstl_hbm.at[c * GPC + p * PG + g + 1].at[s],
                                 bufs[(g + 1) % 2], isem)

            def chunk(j, carry):
                pltpu.async_copy(ones_v, deg_sh.at[ib.at[j]], ssem, add=True)
                return carry

            lax.fori_loop(0, NCH, chunk, 0)

        def drain(j, carry):
            pltpu.make_async_copy(ones_v, deg_sh.at[idx_v.at[0]],
                                  ssem).wait()
            return carry

        lax.fori_loop(0, PG * NCH, drain, 0)
        plsc.subcore_barrier()
        pltpu.sync_copy(deg_sh.at[pl.ds(s * PR, PR)], stage_v)
        base = (c * GPC + p * PG) * NE + s * PR
        pltpu.sync_copy(stage_v, deg_out.at[pl.ds(base, PR)])


def _degrees(dst_l):
    return pl.kernel(
        _deg_body,
        out_type=jax.ShapeDtypeStruct((G * NE,), jnp.float32),
        mesh=_sc_mesh(),
        scratch_types=[
            pltpu.VMEM((NCH, CH), jnp.int32),
            pltpu.VMEM((NCH, CH), jnp.int32),
            pltpu.VMEM((CH,), jnp.float32),
            pltpu.VMEM((PR,), jnp.float32),
            pltpu.VMEM_SHARED((PG * NE,), jnp.float32),
            pltpu.SemaphoreType.DMA,
            pltpu.SemaphoreType.DMA,
        ],
    )(dst_l)


NB = 4


def _gcn_body(xs_hbm, dinv_hbm, srcl_hbm, dstl_hbm, acc_out, osum_out,
              sl_v, sl2_v, dl_v, dl2_v, rows_v, dval_v, stage_v, stagev_v,
              xs_sh, acc_sh, dinv_sh, osum_sh,
              isem, gsem, dsem, ssem, osem):
    c = lax.axis_index("c")
    s = lax.axis_index("s")
    sbufs = [sl_v, sl2_v]
    dbufs = [dl_v, dl2_v]
    for p in range(NPASS):
        base = (c * GPC + p * PG) * NE + s * PR
        pltpu.sync_copy(xs_hbm.at[pl.ds(base, PR)], stage_v)
        pltpu.sync_copy(stage_v, xs_sh.at[pl.ds(s * PR, PR)])
        pltpu.sync_copy(stage_v, acc_sh.at[pl.ds(s * PR, PR)])
        for i in range(PR // 16):
            stagev_v[pl.ds(i * 16, 16)] = jnp.zeros((16,), jnp.float32)
        pltpu.sync_copy(stagev_v, osum_sh.at[pl.ds(s * PR, PR)])
        pltpu.sync_copy(dinv_hbm.at[pl.ds(base, PR)], stagev_v)
        pltpu.sync_copy(stagev_v, dinv_sh.at[pl.ds(s * PR, PR)])
        plsc.subcore_barrier()

        pltpu.async_copy(srcl_hbm.at[c * GPC + p * PG].at[s], sbufs[0], isem)
        pltpu.async_copy(dstl_hbm.at[c * GPC + p * PG].at[s], dbufs[0], isem)
        for g in range(PG):
            sb = sbufs[g % 2]
            db = dbufs[g % 2]
            pltpu.make_async_copy(srcl_hbm.at[0].at[s], sb, isem).wait()
            pltpu.make_async_copy(dstl_hbm.at[0].at[s], db, isem).wait()
            if g + 1 < PG:
                ga_n = c * GPC + p * PG + g + 1
                pltpu.async_copy(srcl_hbm.at[ga_n].at[s],
                                 sbufs[(g + 1) % 2], isem)
                pltpu.async_copy(dstl_hbm.at[ga_n].at[s],
                                 dbufs[(g + 1) % 2], isem)

            pltpu.async_copy(xs_sh.at[sb.at[0]], rows_v.at[0], gsem)
            pltpu.async_copy(dinv_sh.at[db.at[0]], dval_v.at[0], dsem)

            def chunk(j, carry):
                b = j % NB
                pltpu.make_async_copy(xs_sh.at[sb.at[j]], rows_v.at[b],
                                      gsem).wait()
                pltpu.make_async_copy(dinv_sh.at[db.at[j]], dval_v.at[b],
                                      dsem).wait()
                pltpu.async_copy(rows_v.at[b], acc_sh.at[db.at[j]], ssem,
                                 add=True)
                pltpu.async_copy(dval_v.at[b], osum_sh.at[sb.at[j]], osem,
                                 add=True)

                @pl.when(j + 1 < NCH)
                def _():
                    bn = (j + 1) % NB

                    @pl.when(j + 1 >= NB)
                    def _():
                        pltpu.make_async_copy(rows_v.at[bn],
                                              acc_sh.at[db.at[j]],
                                              ssem).wait()
                        pltpu.make_async_copy(dval_v.at[bn],
                                              osum_sh.at[sb.at[j]],
                                              osem).wait()

                    pltpu.async_copy(xs_sh.at[sb.at[j + 1]], rows_v.at[bn],
                                     gsem)
                    pltpu.async_copy(dinv_sh.at[db.at[j + 1]], dval_v.at[bn],
                                     dsem)

                return carry

            lax.fori_loop(0, NCH, chunk, 0)
            for k in range(NB):
                pltpu.make_async_copy(rows_v.at[0], acc_sh.at[db.at[0]],
                                      ssem).wait()
                pltpu.make_async_copy(dval_v.at[0], osum_sh.at[sb.at[0]],
                                      osem).wait()
        plsc.subcore_barrier()
        pltpu.sync_copy(acc_sh.at[pl.ds(s * PR, PR)], stage_v)
        pltpu.sync_copy(stage_v, acc_out.at[pl.ds(base, PR)])
        pltpu.sync_copy(osum_sh.at[pl.ds(s * PR, PR)], stagev_v)
        pltpu.sync_copy(stagev_v, osum_out.at[pl.ds(base, PR)])


def _message_passing(xs_rows, dinv_flat, src_l, dst_l):
    return pl.kernel(
        _gcn_body,
        out_type=(jax.ShapeDtypeStruct((G * NE, F), jnp.float32),
                  jax.ShapeDtypeStruct((G * NE,), jnp.float32)),
        mesh=_sc_mesh(),
        scratch_types=[
            pltpu.VMEM((NCH, CH), jnp.int32),
            pltpu.VMEM((NCH, CH), jnp.int32),
            pltpu.VMEM((NCH, CH), jnp.int32),
            pltpu.VMEM((NCH, CH), jnp.int32),
            pltpu.VMEM((NB, CH, F), jnp.float32),
            pltpu.VMEM((NB, CH), jnp.float32),
            pltpu.VMEM((PR, F), jnp.float32),
            pltpu.VMEM((PR,), jnp.float32),
            pltpu.VMEM_SHARED((PG * NE, F), jnp.float32),
            pltpu.VMEM_SHARED((PG * NE, F), jnp.float32),
            pltpu.VMEM_SHARED((PG * NE,), jnp.float32),
            pltpu.VMEM_SHARED((PG * NE,), jnp.float32),
            pltpu.SemaphoreType.DMA,
            pltpu.SemaphoreType.DMA,
            pltpu.SemaphoreType.DMA,
            pltpu.SemaphoreType.DMA,
            pltpu.SemaphoreType.DMA,
        ],
        compiler_params=pltpu.CompilerParams(use_tc_tiling_on_sc=False),
    )(xs_rows, dinv_flat, src_l, dst_l)


def _prescale_body(deg_ref, x_ref, dinv_ref, xs_ref):
    di = lax.rsqrt(deg_ref[...] + 1.0)
    dinv_ref[...] = di
    dit = jnp.transpose(di[0])
    xs_ref[...] = x_ref[...] * dit[None]


def _prescale(deg3, x3):
    return pl.pallas_call(
        _prescale_body,
        grid=(G,),
        in_specs=[
            pl.BlockSpec((1, 1, NE), lambda g: (g, 0, 0)),
            pl.BlockSpec((1, NE, F), lambda g: (g, 0, 0)),
        ],
        out_specs=[
            pl.BlockSpec((1, 1, NE), lambda g: (g, 0, 0)),
            pl.BlockSpec((1, NE, F), lambda g: (g, 0, 0)),
        ],
        out_shape=(jax.ShapeDtypeStruct((G, 1, NE), jnp.float32),
                   jax.ShapeDtypeStruct((G, NE, F), jnp.float32)),
        compiler_params=pltpu.CompilerParams(
            dimension_semantics=("arbitrary",)),
    )(deg3, x3)


def _final_body(acc_ref, dinv_ref, osum_ref, w1_ref, b1_ref,
                w2_ref, b2_ref, wig_ref, wip_ref, whh_ref, bih_ref, bhh_ref,
                prev_ref, wm1_ref, bm1_ref, wm2_ref, bm2_ref, wm3_ref,
                bm3_ref, out_ref, ge_s):
    g = pl.program_id(0)
    di = dinv_ref[0]
    z = jnp.dot(acc_ref[0], w1_ref[...],
                preferred_element_type=jnp.float32)
    h1 = jnp.maximum(jnp.transpose(di) * z + b1_ref[...], 0.0)
    mask = lax.broadcasted_iota(jnp.int32, (1, NE), 1) < N
    coef = jnp.where(mask, di * osum_ref[0] + di * di, 0.0)
    sv = jnp.dot(coef, h1, preferred_element_type=jnp.float32)
    pooled = b2_ref[...] + jnp.dot(sv, w2_ref[...],
                                   preferred_element_type=jnp.float32) / N
    ge_s[pl.ds(g, 1), :] = pooled

    @pl.when(g == G - 1)
    def _():
        ps = ge_s[...]
        h = jnp.zeros((B, LH), jnp.float32)
        cc = jnp.zeros((B, LH), jnp.float32)
        for t in range(T):
            ge_t = jnp.concatenate([ps[t:t + 1], ps[T + t:T + t + 1]], axis=0)
            pp = prev_ref[:, t, :]
            gates = (jnp.dot(ge_t, wig_ref[...],
                             preferred_element_type=jnp.float32)
                     + jnp.dot(pp, wip_ref[...],
                               preferred_element_type=jnp.float32)
                     + jnp.dot(h, whh_ref[...],
                               preferred_element_type=jnp.float32)
                     + bih_ref[...] + bhh_ref[...])
            ig = jax.nn.sigmoid(gates[:, :LH])
            fg = jax.nn.sigmoid(gates[:, LH:2 * LH])
            gg = jnp.tanh(gates[:, 2 * LH:3 * LH])
            og = jax.nn.sigmoid(gates[:, 3 * LH:])
            cc = fg * cc + ig * gg
            h = og * jnp.tanh(cc)
        z1 = jnp.maximum(jnp.dot(h, wm1_ref[...],
                                 preferred_element_type=jnp.float32)
                         + bm1_ref[...], 0.0)
        z2 = jnp.maximum(jnp.dot(z1, wm2_ref[...],
                                 preferred_element_type=jnp.float32)
                         + bm2_ref[...], 0.0)
        out_ref[...] = (jnp.dot(z2, wm3_ref[...],
                                preferred_element_type=jnp.float32)
                        + bm3_ref[...])


def _finalize(acc3, dinv3, osum3, w1, b1, w2, b2, wig, wip, whh,
              bih, bhh, prev, wm1, bm1, wm2, bm2, wm3, bm3):
    full = lambda shape: pl.BlockSpec(shape, lambda g: tuple(0 for _ in shape))
    return pl.pallas_call(
        _final_body,
        grid=(G,),
        in_specs=[
            pl.BlockSpec((1, NE, F), lambda g: (g, 0, 0)),
            pl.BlockSpec((1, 1, NE), lambda g: (g, 0, 0)),
            pl.BlockSpec((1, 1, NE), lambda g: (g, 0, 0)),
            full((F, GH)), full((1, GH)),
            full((GH, GH)), full((1, GH)),
            full((GH, 4 * LH)), full((3, 4 * LH)), full((LH, 4 * LH)),
            full((1, 4 * LH)), full((1, 4 * LH)),
            full((B, T, 3)),
            full((LH, MH)), full((1, MH)),
            full((MH, MH)), full((1, MH)),
            full((MH, 3)), full((1, 3)),
        ],
        out_specs=pl.BlockSpec((B, 3), lambda g: (0, 0)),
        out_shape=jax.ShapeDtypeStruct((B, 3), jnp.float32),
        scratch_shapes=[pltpu.VMEM((G, GH), jnp.float32)],
        compiler_params=pltpu.CompilerParams(
            dimension_semantics=("arbitrary",)),
    )(acc3, dinv3, osum3, w1, b1, w2, b2, wig, wip, whh, bih, bhh,
      prev, wm1, bm1, wm2, bm2, wm3, bm3)


def kernel(node_feats_seq, edge_index_seq, prev_pos_seq,
           W_gcn1, b_gcn1, W_gcn2, b_gcn2,
           W_ih, W_hh, b_ih, b_hh,
           W_m1, b_m1, W_m2, b_m2, W_m3, b_m3):
    f32 = jnp.float32
    ei = edge_index_seq.astype(jnp.int32).reshape(G, 2, E)
    pad = N + (jnp.arange(EP - E, dtype=jnp.int32) % NPAD)
    pad = jnp.broadcast_to(pad, (G, EP - E))
    src = jnp.concatenate([ei[:, 0], pad], axis=1)
    dst = jnp.concatenate([ei[:, 1], pad], axis=1)
    loff = ((jnp.arange(G, dtype=jnp.int32) % PG) * NE)[:, None]
    shp = (G, NT, NCH, CH)
    src_l = (src + loff).reshape(shp)
    dst_l = (dst + loff).reshape(shp)

    deg = _degrees(dst_l)

    x3 = jnp.pad(node_feats_seq.reshape(G, N, F), ((0, 0), (0, NPAD), (0, 0)))
    dinv3, xs3 = _prescale(deg.reshape(G, 1, NE), x3)

    acc, osum = _message_passing(
        xs3.reshape(G * NE, F), dinv3.reshape(G * NE), src_l, dst_l)

    wih_t = jnp.transpose(W_ih)
    pred = _finalize(
        acc.reshape(G, NE, F), dinv3, osum.reshape(G, 1, NE),
        W_gcn1, b_gcn1.reshape(1, GH), W_gcn2, b_gcn2.reshape(1, GH),
        wih_t[:GH], wih_t[GH:], jnp.transpose(W_hh),
        b_ih.reshape(1, 4 * LH), b_hh.reshape(1, 4 * LH),
        prev_pos_seq,
        jnp.transpose(W_m1), b_m1.reshape(1, MH),
        jnp.transpose(W_m2), b_m2.reshape(1, MH),
        jnp.transpose(W_m3), b_m3.reshape(1, 3))
    return pred
